# baseline (speedup 1.0000x reference)
import functools

import jax
import jax.numpy as jnp
from jax import lax
from jax.experimental import pallas as pl
from jax.experimental.pallas import tpu as pltpu
from jax.experimental.pallas import tpu_sc as plsc

NC, NS, L = 2, 16, 16
NW = NC * NS
CHUNK = 128
POOL_ROWS = 272
DUMP = 256

BLK = 2048


def _sc_segsum_body(feat_hbm, batch_hbm, out_hbm, rows_v, idx_v, zb_v, tmp_v,
                    pool_sh, *, n_rows, nchunks, per_w):
    c = lax.axis_index("c")
    s = lax.axis_index("s")
    w = s * NC + c

    for i in range(17):
        for j in range(8):
            zb_v[i, pl.ds(j * L, L)] = jnp.zeros((L,), jnp.float32)
    pltpu.sync_copy(zb_v, pool_sh.at[pl.ds(s * 17, 17)])
    plsc.subcore_barrier()

    def step(t, carry):
        g = t * NW + w

        @pl.when(g < nchunks)
        def _do():
            base = g * CHUNK
            base_eff = jnp.minimum(base, n_rows - CHUNK)
            pltpu.sync_copy(feat_hbm.at[pl.ds(base_eff, CHUNK)], rows_v)
            pltpu.sync_copy(batch_hbm.at[pl.ds(base_eff, CHUNK)], idx_v)
            lane = lax.broadcasted_iota(jnp.int32, (L,), 0)
            for j in range(CHUNK // L):
                rid = base_eff + j * L + lane
                v = idx_v[pl.ds(j * L, L)]
                idx_v[pl.ds(j * L, L)] = jnp.where(rid >= base, v, DUMP)
            pltpu.sync_copy(rows_v, pool_sh.at[idx_v], add=True)

        return carry

    lax.fori_loop(0, per_w, step, 0)
    plsc.subcore_barrier()

    pltpu.sync_copy(pool_sh.at[pl.ds(s * L, L)], tmp_v)
    pltpu.sync_copy(tmp_v, out_hbm.at[c, pl.ds(s * L, L)])


def _sc_segsum(feat, batch):
    n, d = feat.shape
    nchunks = (n + CHUNK - 1) // CHUNK
    per_w = (nchunks + NW - 1) // NW
    mesh = plsc.VectorSubcoreMesh(core_axis_name="c", subcore_axis_name="s",
                                  num_cores=NC, num_subcores=NS)
    f = pl.kernel(
        functools.partial(_sc_segsum_body, n_rows=n, nchunks=nchunks,
                          per_w=per_w),
        out_type=jax.ShapeDtypeStruct((NC, NS * L, d), jnp.float32),
        mesh=mesh,
        scratch_types=[
            pltpu.VMEM((CHUNK, d), jnp.float32),
            pltpu.VMEM((CHUNK,), jnp.int32),
            pltpu.VMEM((17, d), jnp.float32),
            pltpu.VMEM((L, d), jnp.float32),
            pltpu.VMEM_SHARED((POOL_ROWS, d), jnp.float32),
        ],
    )
    return f(feat, batch)


def _tc_body(feat_ref, batch_ref, part_ref, vn_ref, w_ref, b_ref,
             out_ref, vnout_ref, vn_scr, *, num_graphs):
    i = pl.program_id(0)

    @pl.when(i == 0)
    def _fc():
        pool = part_ref[0] + part_ref[1]
        vn_tmp = pool + vn_ref[...]
        vn_o = jnp.maximum(
            jnp.dot(vn_tmp, w_ref[...], preferred_element_type=jnp.float32)
            + b_ref[...], 0.0) + vn_ref[...]
        vn_scr[...] = vn_o
        vnout_ref[...] = vn_o

    bvec = batch_ref[0, 0, :]
    gids = lax.broadcasted_iota(jnp.int32, (BLK, num_graphs), 1)
    onehot = jnp.where(gids == bvec[:, None], 1.0, 0.0)
    out_ref[...] = feat_ref[...] + jnp.dot(
        onehot, vn_scr[...], preferred_element_type=jnp.float32)


def kernel(feat, vn_feat, W, b, batch):
    n, d = feat.shape
    num_graphs = vn_feat.shape[0]

    partials = _sc_segsum(feat, batch)

    num_blocks = (n + BLK - 1) // BLK
    pad = num_blocks * BLK - n
    batch_r = jnp.pad(batch, (0, pad)).reshape(num_blocks, 1, BLK)

    feat_out, vn_out = pl.pallas_call(
        functools.partial(_tc_body, num_graphs=num_graphs),
        grid=(num_blocks,),
        in_specs=[
            pl.BlockSpec((BLK, d), lambda i: (i, 0)),
            pl.BlockSpec((1, 1, BLK), lambda i: (i, 0, 0)),
            pl.BlockSpec((NC, num_graphs, d), lambda i: (0, 0, 0)),
            pl.BlockSpec((num_graphs, d), lambda i: (0, 0)),
            pl.BlockSpec((d, d), lambda i: (0, 0)),
            pl.BlockSpec((1, d), lambda i: (0, 0)),
        ],
        out_specs=[
            pl.BlockSpec((BLK, d), lambda i: (i, 0)),
            pl.BlockSpec((num_graphs, d), lambda i: (0, 0)),
        ],
        scratch_shapes=[
            pltpu.VMEM((num_graphs, d), jnp.float32),
        ],
        out_shape=(
            jax.ShapeDtypeStruct((n, d), jnp.float32),
            jax.ShapeDtypeStruct((num_graphs, d), jnp.float32),
        ),
        compiler_params=pltpu.CompilerParams(
            dimension_semantics=("arbitrary",),
        ),
    )(feat, batch_r, partials, vn_feat, W, b.reshape(1, d))
    return (feat_out, vn_out)


# speedup vs baseline: 1.4160x; 1.4160x over previous
import functools

import jax
import jax.numpy as jnp
from jax import lax
from jax.experimental import pallas as pl
from jax.experimental.pallas import tpu as pltpu
from jax.experimental.pallas import tpu_sc as plsc

NC, NS, L = 2, 16, 16
NW = NC * NS
CHUNK = 128
NBUF = 2
POOL_ROWS = 272
DUMP = 256

BLK = 2048
N1_BLOCKS = 22


def _sc_segsum_body(feat_hbm, batch_hbm, out_hbm, rows_v, idx_v, zb_v, tmp_v,
                    pool_sh, sem0, sem1, *, row0, n_rows, nchunks, per_w):
    c = lax.axis_index("c")
    s = lax.axis_index("s")
    w = s * NC + c
    sems = (sem0, sem1)

    for i in range(17):
        for j in range(8):
            zb_v[i, pl.ds(j * L, L)] = jnp.zeros((L,), jnp.float32)
    pltpu.sync_copy(zb_v, pool_sh.at[pl.ds(s * 17, 17)])
    plsc.subcore_barrier()

    def base_of(g):
        return row0 + jnp.minimum(g * CHUNK, (n_rows - row0) - CHUNK)

    def start(t, b):
        g = t * NW + w

        @pl.when(g < nchunks)
        def _():
            be = base_of(g)
            pltpu.async_copy(feat_hbm.at[pl.ds(be, CHUNK)], rows_v.at[b],
                             sems[b])
            pltpu.async_copy(batch_hbm.at[pl.ds(be, CHUNK)], idx_v.at[b],
                             sems[b])

    def finish(t, b):
        g = t * NW + w

        @pl.when(g < nchunks)
        def _():
            base = row0 + g * CHUNK
            be = base_of(g)
            pltpu.make_async_copy(feat_hbm.at[pl.ds(be, CHUNK)],
                                  rows_v.at[b], sems[b]).wait()
            pltpu.make_async_copy(batch_hbm.at[pl.ds(be, CHUNK)],
                                  idx_v.at[b], sems[b]).wait()
            lane = lax.broadcasted_iota(jnp.int32, (L,), 0)
            for j in range(CHUNK // L):
                rid = be + j * L + lane
                v = idx_v[b, pl.ds(j * L, L)]
                idx_v[b, pl.ds(j * L, L)] = jnp.where(rid >= base, v, DUMP)
            pltpu.sync_copy(rows_v.at[b], pool_sh.at[idx_v.at[b]], add=True)

    for b in range(NBUF):
        start(b, b)

    ngroups = (per_w + NBUF - 1) // NBUF

    def gbody(tg, carry):
        for b in range(NBUF):
            t = tg * NBUF + b
            finish(t, b)
            start(t + NBUF, b)
        return carry

    lax.fori_loop(0, ngroups, gbody, 0)
    plsc.subcore_barrier()

    pltpu.sync_copy(pool_sh.at[pl.ds(s * L, L)], tmp_v)
    pltpu.sync_copy(tmp_v, out_hbm.at[c, pl.ds(s * L, L)])


def _sc_segsum(feat, batch, row0):
    n, d = feat.shape
    nchunks = (n - row0 + CHUNK - 1) // CHUNK
    per_w = (nchunks + NW - 1) // NW
    mesh = plsc.VectorSubcoreMesh(core_axis_name="c", subcore_axis_name="s",
                                  num_cores=NC, num_subcores=NS)
    f = pl.kernel(
        functools.partial(_sc_segsum_body, row0=row0, n_rows=n,
                          nchunks=nchunks, per_w=per_w),
        out_type=jax.ShapeDtypeStruct((NC, NS * L, d), jnp.float32),
        mesh=mesh,
        scratch_types=[
            pltpu.VMEM((NBUF, CHUNK, d), jnp.float32),
            pltpu.VMEM((NBUF, CHUNK), jnp.int32),
            pltpu.VMEM((17, d), jnp.float32),
            pltpu.VMEM((L, d), jnp.float32),
            pltpu.VMEM_SHARED((POOL_ROWS, d), jnp.float32),
            pltpu.SemaphoreType.DMA,
            pltpu.SemaphoreType.DMA,
        ],
    )
    return f(feat, batch)


def _tc_pool_body(feat_ref, batch_ref, out_ref, *, num_graphs):
    i = pl.program_id(0)

    @pl.when(i == 0)
    def _init():
        out_ref[...] = jnp.zeros_like(out_ref)

    bvec = batch_ref[0, 0, :]
    gids = lax.broadcasted_iota(jnp.int32, (num_graphs, BLK), 0)
    onehot_t = jnp.where(gids == bvec[None, :], 1.0, 0.0)
    out_ref[...] += jnp.dot(onehot_t, feat_ref[...],
                            preferred_element_type=jnp.float32)


def _tc_bc_body(feat_ref, batch_ref, psc_ref, ptc_ref, vn_ref, w_ref, b_ref,
                out_ref, vnout_ref, vn_scr, *, num_graphs):
    i = pl.program_id(0)

    @pl.when(i == 0)
    def _fc():
        pool = psc_ref[0] + psc_ref[1] + ptc_ref[...]
        vn_tmp = pool + vn_ref[...]
        vn_o = jnp.maximum(
            jnp.dot(vn_tmp, w_ref[...], preferred_element_type=jnp.float32)
            + b_ref[...], 0.0) + vn_ref[...]
        vn_scr[...] = vn_o
        vnout_ref[...] = vn_o

    bvec = batch_ref[0, 0, :]
    gids = lax.broadcasted_iota(jnp.int32, (BLK, num_graphs), 1)
    onehot = jnp.where(gids == bvec[:, None], 1.0, 0.0)
    out_ref[...] = feat_ref[...] + jnp.dot(
        onehot, vn_scr[...], preferred_element_type=jnp.float32)


def kernel(feat, vn_feat, W, b, batch):
    n, d = feat.shape
    num_graphs = vn_feat.shape[0]
    n1 = N1_BLOCKS * BLK

    num_blocks = (n + BLK - 1) // BLK
    pad = num_blocks * BLK - n
    batch_r = jnp.pad(batch, (0, pad)).reshape(num_blocks, 1, BLK)

    p_sc = _sc_segsum(feat, batch, n1)

    p_tc = pl.pallas_call(
        functools.partial(_tc_pool_body, num_graphs=num_graphs),
        grid=(N1_BLOCKS,),
        in_specs=[
            pl.BlockSpec((BLK, d), lambda i: (i, 0)),
            pl.BlockSpec((1, 1, BLK), lambda i: (i, 0, 0)),
        ],
        out_specs=pl.BlockSpec((num_graphs, d), lambda i: (0, 0)),
        out_shape=jax.ShapeDtypeStruct((num_graphs, d), jnp.float32),
        compiler_params=pltpu.CompilerParams(
            dimension_semantics=("arbitrary",),
        ),
    )(feat, batch_r)

    feat_out, vn_out = pl.pallas_call(
        functools.partial(_tc_bc_body, num_graphs=num_graphs),
        grid=(num_blocks,),
        in_specs=[
            pl.BlockSpec((BLK, d), lambda i: (i, 0)),
            pl.BlockSpec((1, 1, BLK), lambda i: (i, 0, 0)),
            pl.BlockSpec((NC, num_graphs, d), lambda i: (0, 0, 0)),
            pl.BlockSpec((num_graphs, d), lambda i: (0, 0)),
            pl.BlockSpec((num_graphs, d), lambda i: (0, 0)),
            pl.BlockSpec((d, d), lambda i: (0, 0)),
            pl.BlockSpec((1, d), lambda i: (0, 0)),
        ],
        out_specs=[
            pl.BlockSpec((BLK, d), lambda i: (i, 0)),
            pl.BlockSpec((num_graphs, d), lambda i: (0, 0)),
        ],
        scratch_shapes=[
            pltpu.VMEM((num_graphs, d), jnp.float32),
        ],
        out_shape=(
            jax.ShapeDtypeStruct((n, d), jnp.float32),
            jax.ShapeDtypeStruct((num_graphs, d), jnp.float32),
        ),
        compiler_params=pltpu.CompilerParams(
            dimension_semantics=("arbitrary",),
        ),
    )(feat, batch_r, p_sc, p_tc, vn_feat, W, b.reshape(1, d))
    return (feat_out, vn_out)


# speedup vs baseline: 1.7138x; 1.2103x over previous
import functools

import jax
import jax.numpy as jnp
from jax import lax
from jax.experimental import pallas as pl
from jax.experimental.pallas import tpu as pltpu
from jax.experimental.pallas import tpu_sc as plsc

NC, NS, L = 2, 16, 16
NW = NC * NS
CHUNK = 128
NBUF = 3
POOL_ROWS = 272
DUMP = 256

BLK = 4096
N1_BLOCKS = 11


def _sc_segsum_body(feat_hbm, batch_hbm, out_hbm, rows_v, idx_v, zb_v, tmp_v,
                    pool_sh, sem0, sem1, sem2, *, row0, n_rows, nchunks,
                    per_w):
    c = lax.axis_index("c")
    s = lax.axis_index("s")
    w = s * NC + c
    sems = (sem0, sem1, sem2)

    for i in range(17):
        for j in range(8):
            zb_v[i, pl.ds(j * L, L)] = jnp.zeros((L,), jnp.float32)
    pltpu.sync_copy(zb_v, pool_sh.at[pl.ds(s * 17, 17)])
    plsc.subcore_barrier()

    def base_of(g):
        return row0 + jnp.minimum(g * CHUNK, (n_rows - row0) - CHUNK)

    def start(t, b):
        g = t * NW + w

        @pl.when(g < nchunks)
        def _():
            be = base_of(g)
            pltpu.async_copy(feat_hbm.at[pl.ds(be, CHUNK)], rows_v.at[b],
                             sems[b])
            pltpu.async_copy(batch_hbm.at[pl.ds(be, CHUNK)], idx_v.at[b],
                             sems[b])

    def finish(t, b):
        g = t * NW + w

        @pl.when(g < nchunks)
        def _():
            base = row0 + g * CHUNK
            be = base_of(g)
            pltpu.make_async_copy(feat_hbm.at[pl.ds(be, CHUNK)],
                                  rows_v.at[b], sems[b]).wait()
            pltpu.make_async_copy(batch_hbm.at[pl.ds(be, CHUNK)],
                                  idx_v.at[b], sems[b]).wait()
            lane = lax.broadcasted_iota(jnp.int32, (L,), 0)
            for j in range(CHUNK // L):
                rid = be + j * L + lane
                v = idx_v[b, pl.ds(j * L, L)]
                idx_v[b, pl.ds(j * L, L)] = jnp.where(rid >= base, v, DUMP)
            pltpu.sync_copy(rows_v.at[b], pool_sh.at[idx_v.at[b]], add=True)

    for b in range(NBUF):
        start(b, b)

    ngroups = (per_w + NBUF - 1) // NBUF

    def gbody(tg, carry):
        for b in range(NBUF):
            t = tg * NBUF + b
            finish(t, b)
            start(t + NBUF, b)
        return carry

    lax.fori_loop(0, ngroups, gbody, 0)
    plsc.subcore_barrier()

    pltpu.sync_copy(pool_sh.at[pl.ds(s * L, L)], tmp_v)
    pltpu.sync_copy(tmp_v, out_hbm.at[c, pl.ds(s * L, L)])


def _sc_segsum(feat, batch, row0):
    n, d = feat.shape
    nchunks = (n - row0 + CHUNK - 1) // CHUNK
    per_w = (nchunks + NW - 1) // NW
    mesh = plsc.VectorSubcoreMesh(core_axis_name="c", subcore_axis_name="s",
                                  num_cores=NC, num_subcores=NS)
    f = pl.kernel(
        functools.partial(_sc_segsum_body, row0=row0, n_rows=n,
                          nchunks=nchunks, per_w=per_w),
        out_type=jax.ShapeDtypeStruct((NC, NS * L, d), jnp.float32),
        mesh=mesh,
        scratch_types=[
            pltpu.VMEM((NBUF, CHUNK, d), jnp.float32),
            pltpu.VMEM((NBUF, CHUNK), jnp.int32),
            pltpu.VMEM((17, d), jnp.float32),
            pltpu.VMEM((L, d), jnp.float32),
            pltpu.VMEM_SHARED((POOL_ROWS, d), jnp.float32),
            pltpu.SemaphoreType.DMA,
            pltpu.SemaphoreType.DMA,
            pltpu.SemaphoreType.DMA,
        ],
    )
    return f(feat, batch)


def _tc_pool_body(feat_ref, batch_ref, out_ref, *, num_graphs):
    i = pl.program_id(0)

    @pl.when(i == 0)
    def _init():
        out_ref[...] = jnp.zeros_like(out_ref)

    bvec = batch_ref[0, 0, :]
    gids = lax.broadcasted_iota(jnp.int32, (num_graphs, BLK), 0)
    onehot_t = jnp.where(gids == bvec[None, :], 1.0, 0.0)
    out_ref[...] += jnp.dot(onehot_t, feat_ref[...],
                            preferred_element_type=jnp.float32)


def _tc_bc_body(feat_ref, batch_ref, psc_ref, ptc_ref, vn_ref, w_ref, b_ref,
                out_ref, vnout_ref, vn_scr, *, num_graphs):
    i = pl.program_id(0)

    @pl.when(i == 0)
    def _fc():
        pool = psc_ref[0] + psc_ref[1] + ptc_ref[...]
        vn_tmp = pool + vn_ref[...]
        vn_o = jnp.maximum(
            jnp.dot(vn_tmp, w_ref[...], preferred_element_type=jnp.float32)
            + b_ref[...], 0.0) + vn_ref[...]
        vn_scr[...] = vn_o
        vnout_ref[...] = vn_o

    bvec = batch_ref[0, 0, :]
    gids = lax.broadcasted_iota(jnp.int32, (BLK, num_graphs), 1)
    onehot = jnp.where(gids == bvec[:, None], 1.0, 0.0)
    out_ref[...] = feat_ref[...] + jnp.dot(
        onehot, vn_scr[...], preferred_element_type=jnp.float32)


def kernel(feat, vn_feat, W, b, batch):
    n, d = feat.shape
    num_graphs = vn_feat.shape[0]
    n1 = N1_BLOCKS * BLK

    num_blocks = (n + BLK - 1) // BLK
    pad = num_blocks * BLK - n
    batch_r = jnp.pad(batch, (0, pad)).reshape(num_blocks, 1, BLK)

    p_sc = _sc_segsum(feat, batch, n1)

    p_tc = pl.pallas_call(
        functools.partial(_tc_pool_body, num_graphs=num_graphs),
        grid=(N1_BLOCKS,),
        in_specs=[
            pl.BlockSpec((BLK, d), lambda i: (i, 0)),
            pl.BlockSpec((1, 1, BLK), lambda i: (i, 0, 0)),
        ],
        out_specs=pl.BlockSpec((num_graphs, d), lambda i: (0, 0)),
        out_shape=jax.ShapeDtypeStruct((num_graphs, d), jnp.float32),
        compiler_params=pltpu.CompilerParams(
            dimension_semantics=("arbitrary",),
        ),
    )(feat, batch_r)

    feat_out, vn_out = pl.pallas_call(
        functools.partial(_tc_bc_body, num_graphs=num_graphs),
        grid=(num_blocks,),
        in_specs=[
            pl.BlockSpec((BLK, d), lambda i: (i, 0)),
            pl.BlockSpec((1, 1, BLK), lambda i: (i, 0, 0)),
            pl.BlockSpec((NC, num_graphs, d), lambda i: (0, 0, 0)),
            pl.BlockSpec((num_graphs, d), lambda i: (0, 0)),
            pl.BlockSpec((num_graphs, d), lambda i: (0, 0)),
            pl.BlockSpec((d, d), lambda i: (0, 0)),
            pl.BlockSpec((1, d), lambda i: (0, 0)),
        ],
        out_specs=[
            pl.BlockSpec((BLK, d), lambda i: (i, 0)),
            pl.BlockSpec((num_graphs, d), lambda i: (0, 0)),
        ],
        scratch_shapes=[
            pltpu.VMEM((num_graphs, d), jnp.float32),
        ],
        out_shape=(
            jax.ShapeDtypeStruct((n, d), jnp.float32),
            jax.ShapeDtypeStruct((num_graphs, d), jnp.float32),
        ),
        compiler_params=pltpu.CompilerParams(
            dimension_semantics=("arbitrary",),
        ),
    )(feat, batch_r, p_sc, p_tc, vn_feat, W, b.reshape(1, d))
    return (feat_out, vn_out)


# speedup vs baseline: 1.8636x; 1.0874x over previous
import functools

import jax
import jax.numpy as jnp
from jax import lax
from jax.experimental import pallas as pl
from jax.experimental.pallas import tpu as pltpu
from jax.experimental.pallas import tpu_sc as plsc

NC, NS, L = 2, 16, 16
NW = NC * NS
CHUNK = 128
NBUF = 3
POOL_ROWS = 272
DUMP = 256

BLK = 8192
N1_BLOCKS = 6


def _sc_segsum_body(feat_hbm, batch_hbm, out_hbm, rows_v, idx_v, zb_v, tmp_v,
                    pool_sh, sem0, sem1, sem2, *, row0, n_rows, nchunks,
                    per_w):
    c = lax.axis_index("c")
    s = lax.axis_index("s")
    w = s * NC + c
    sems = (sem0, sem1, sem2)

    for i in range(17):
        for j in range(8):
            zb_v[i, pl.ds(j * L, L)] = jnp.zeros((L,), jnp.float32)
    pltpu.sync_copy(zb_v, pool_sh.at[pl.ds(s * 17, 17)])
    plsc.subcore_barrier()

    def base_of(g):
        return row0 + jnp.minimum(g * CHUNK, (n_rows - row0) - CHUNK)

    def start(t, b):
        g = t * NW + w

        @pl.when(g < nchunks)
        def _():
            be = base_of(g)
            pltpu.async_copy(feat_hbm.at[pl.ds(be, CHUNK)], rows_v.at[b],
                             sems[b])
            pltpu.async_copy(batch_hbm.at[pl.ds(be, CHUNK)], idx_v.at[b],
                             sems[b])

    def finish(t, b):
        g = t * NW + w

        @pl.when(g < nchunks)
        def _():
            base = row0 + g * CHUNK
            be = base_of(g)
            pltpu.make_async_copy(feat_hbm.at[pl.ds(be, CHUNK)],
                                  rows_v.at[b], sems[b]).wait()
            pltpu.make_async_copy(batch_hbm.at[pl.ds(be, CHUNK)],
                                  idx_v.at[b], sems[b]).wait()
            lane = lax.broadcasted_iota(jnp.int32, (L,), 0)
            for j in range(CHUNK // L):
                rid = be + j * L + lane
                v = idx_v[b, pl.ds(j * L, L)]
                idx_v[b, pl.ds(j * L, L)] = jnp.where(rid >= base, v, DUMP)
            pltpu.sync_copy(rows_v.at[b], pool_sh.at[idx_v.at[b]], add=True)

    for b in range(NBUF):
        start(b, b)

    ngroups = (per_w + NBUF - 1) // NBUF

    def gbody(tg, carry):
        for b in range(NBUF):
            t = tg * NBUF + b
            finish(t, b)
            start(t + NBUF, b)
        return carry

    lax.fori_loop(0, ngroups, gbody, 0)
    plsc.subcore_barrier()

    pltpu.sync_copy(pool_sh.at[pl.ds(s * L, L)], tmp_v)
    pltpu.sync_copy(tmp_v, out_hbm.at[c, pl.ds(s * L, L)])


def _sc_segsum(feat, batch, row0):
    n, d = feat.shape
    nchunks = (n - row0 + CHUNK - 1) // CHUNK
    per_w = (nchunks + NW - 1) // NW
    mesh = plsc.VectorSubcoreMesh(core_axis_name="c", subcore_axis_name="s",
                                  num_cores=NC, num_subcores=NS)
    f = pl.kernel(
        functools.partial(_sc_segsum_body, row0=row0, n_rows=n,
                          nchunks=nchunks, per_w=per_w),
        out_type=jax.ShapeDtypeStruct((NC, NS * L, d), jnp.float32),
        mesh=mesh,
        scratch_types=[
            pltpu.VMEM((NBUF, CHUNK, d), jnp.float32),
            pltpu.VMEM((NBUF, CHUNK), jnp.int32),
            pltpu.VMEM((17, d), jnp.float32),
            pltpu.VMEM((L, d), jnp.float32),
            pltpu.VMEM_SHARED((POOL_ROWS, d), jnp.float32),
            pltpu.SemaphoreType.DMA,
            pltpu.SemaphoreType.DMA,
            pltpu.SemaphoreType.DMA,
        ],
    )
    return f(feat, batch)


def _tc_pool_body(feat_ref, batch_ref, out_ref, *, num_graphs):
    i = pl.program_id(0)

    @pl.when(i == 0)
    def _init():
        out_ref[...] = jnp.zeros_like(out_ref)

    bvec = batch_ref[0, 0, :]
    gids = lax.broadcasted_iota(jnp.int32, (num_graphs, BLK), 0)
    onehot_t = jnp.where(gids == bvec[None, :], 1.0, 0.0)
    out_ref[...] += jnp.dot(onehot_t, feat_ref[...],
                            preferred_element_type=jnp.float32)


def _tc_bc_body(feat_ref, batch_ref, psc_ref, ptc_ref, vn_ref, w_ref, b_ref,
                out_ref, vnout_ref, vn_scr, *, num_graphs):
    i = pl.program_id(0)

    @pl.when(i == 0)
    def _fc():
        pool = psc_ref[0] + psc_ref[1] + ptc_ref[...]
        vn_tmp = pool + vn_ref[...]
        vn_o = jnp.maximum(
            jnp.dot(vn_tmp, w_ref[...], preferred_element_type=jnp.float32)
            + b_ref[...], 0.0) + vn_ref[...]
        vn_scr[...] = vn_o
        vnout_ref[...] = vn_o

    bvec = batch_ref[0, 0, :]
    gids = lax.broadcasted_iota(jnp.int32, (BLK, num_graphs), 1)
    onehot = jnp.where(gids == bvec[:, None], 1.0, 0.0)
    out_ref[...] = feat_ref[...] + jnp.dot(
        onehot, vn_scr[...], preferred_element_type=jnp.float32)


def kernel(feat, vn_feat, W, b, batch):
    n, d = feat.shape
    num_graphs = vn_feat.shape[0]
    n1 = N1_BLOCKS * BLK

    num_blocks = (n + BLK - 1) // BLK
    pad = num_blocks * BLK - n
    batch_r = jnp.pad(batch, (0, pad)).reshape(num_blocks, 1, BLK)

    p_sc = _sc_segsum(feat, batch, n1)

    p_tc = pl.pallas_call(
        functools.partial(_tc_pool_body, num_graphs=num_graphs),
        grid=(N1_BLOCKS,),
        in_specs=[
            pl.BlockSpec((BLK, d), lambda i: (i, 0)),
            pl.BlockSpec((1, 1, BLK), lambda i: (i, 0, 0)),
        ],
        out_specs=pl.BlockSpec((num_graphs, d), lambda i: (0, 0)),
        out_shape=jax.ShapeDtypeStruct((num_graphs, d), jnp.float32),
        compiler_params=pltpu.CompilerParams(
            dimension_semantics=("arbitrary",),
        ),
    )(feat, batch_r)

    feat_out, vn_out = pl.pallas_call(
        functools.partial(_tc_bc_body, num_graphs=num_graphs),
        grid=(num_blocks,),
        in_specs=[
            pl.BlockSpec((BLK, d), lambda i: (i, 0)),
            pl.BlockSpec((1, 1, BLK), lambda i: (i, 0, 0)),
            pl.BlockSpec((NC, num_graphs, d), lambda i: (0, 0, 0)),
            pl.BlockSpec((num_graphs, d), lambda i: (0, 0)),
            pl.BlockSpec((num_graphs, d), lambda i: (0, 0)),
            pl.BlockSpec((d, d), lambda i: (0, 0)),
            pl.BlockSpec((1, d), lambda i: (0, 0)),
        ],
        out_specs=[
            pl.BlockSpec((BLK, d), lambda i: (i, 0)),
            pl.BlockSpec((num_graphs, d), lambda i: (0, 0)),
        ],
        scratch_shapes=[
            pltpu.VMEM((num_graphs, d), jnp.float32),
        ],
        out_shape=(
            jax.ShapeDtypeStruct((n, d), jnp.float32),
            jax.ShapeDtypeStruct((num_graphs, d), jnp.float32),
        ),
        compiler_params=pltpu.CompilerParams(
            dimension_semantics=("arbitrary",),
        ),
    )(feat, batch_r, p_sc, p_tc, vn_feat, W, b.reshape(1, d))
    return (feat_out, vn_out)


# speedup vs baseline: 1.9167x; 1.0285x over previous
import functools

import jax
import jax.numpy as jnp
from jax import lax
from jax.experimental import pallas as pl
from jax.experimental.pallas import tpu as pltpu
from jax.experimental.pallas import tpu_sc as plsc

NC, NS, L = 2, 16, 16
NW = NC * NS
CHUNK = 128
NBUF = 3
POOL_ROWS = 272
DUMP = 256

BLK = 8192
N1_BLOCKS = 7


def _sc_segsum_body(feat_hbm, batch_hbm, out_hbm, rows_v, idx_v, zb_v, tmp_v,
                    pool_sh, sem0, sem1, sem2, *, row0, n_rows, nchunks,
                    per_w):
    c = lax.axis_index("c")
    s = lax.axis_index("s")
    w = s * NC + c
    sems = (sem0, sem1, sem2)

    for i in range(17):
        for j in range(8):
            zb_v[i, pl.ds(j * L, L)] = jnp.zeros((L,), jnp.float32)
    pltpu.sync_copy(zb_v, pool_sh.at[pl.ds(s * 17, 17)])
    plsc.subcore_barrier()

    def base_of(g):
        return row0 + jnp.minimum(g * CHUNK, (n_rows - row0) - CHUNK)

    def start(t, b):
        g = t * NW + w

        @pl.when(g < nchunks)
        def _():
            be = base_of(g)
            pltpu.async_copy(feat_hbm.at[pl.ds(be, CHUNK)], rows_v.at[b],
                             sems[b])
            pltpu.async_copy(batch_hbm.at[pl.ds(be, CHUNK)], idx_v.at[b],
                             sems[b])

    def finish(t, b):
        g = t * NW + w

        @pl.when(g < nchunks)
        def _():
            base = row0 + g * CHUNK
            be = base_of(g)
            pltpu.make_async_copy(feat_hbm.at[pl.ds(be, CHUNK)],
                                  rows_v.at[b], sems[b]).wait()
            pltpu.make_async_copy(batch_hbm.at[pl.ds(be, CHUNK)],
                                  idx_v.at[b], sems[b]).wait()
            lane = lax.broadcasted_iota(jnp.int32, (L,), 0)
            for j in range(CHUNK // L):
                rid = be + j * L + lane
                v = idx_v[b, pl.ds(j * L, L)]
                idx_v[b, pl.ds(j * L, L)] = jnp.where(rid >= base, v, DUMP)
            pltpu.sync_copy(rows_v.at[b], pool_sh.at[idx_v.at[b]], add=True)

    for b in range(NBUF):
        start(b, b)

    ngroups = (per_w + NBUF - 1) // NBUF

    def gbody(tg, carry):
        for b in range(NBUF):
            t = tg * NBUF + b
            finish(t, b)
            start(t + NBUF, b)
        return carry

    lax.fori_loop(0, ngroups, gbody, 0)
    plsc.subcore_barrier()

    pltpu.sync_copy(pool_sh.at[pl.ds(s * L, L)], tmp_v)
    pltpu.sync_copy(tmp_v, out_hbm.at[c, pl.ds(s * L, L)])


def _sc_segsum(feat, batch, row0):
    n, d = feat.shape
    nchunks = (n - row0 + CHUNK - 1) // CHUNK
    per_w = (nchunks + NW - 1) // NW
    mesh = plsc.VectorSubcoreMesh(core_axis_name="c", subcore_axis_name="s",
                                  num_cores=NC, num_subcores=NS)
    f = pl.kernel(
        functools.partial(_sc_segsum_body, row0=row0, n_rows=n,
                          nchunks=nchunks, per_w=per_w),
        out_type=jax.ShapeDtypeStruct((NC, NS * L, d), jnp.float32),
        mesh=mesh,
        scratch_types=[
            pltpu.VMEM((NBUF, CHUNK, d), jnp.float32),
            pltpu.VMEM((NBUF, CHUNK), jnp.int32),
            pltpu.VMEM((17, d), jnp.float32),
            pltpu.VMEM((L, d), jnp.float32),
            pltpu.VMEM_SHARED((POOL_ROWS, d), jnp.float32),
            pltpu.SemaphoreType.DMA,
            pltpu.SemaphoreType.DMA,
            pltpu.SemaphoreType.DMA,
        ],
    )
    return f(feat, batch)


def _tc_pool_body(feat_ref, batch_ref, out_ref, *, num_graphs):
    i = pl.program_id(0)

    @pl.when(i == 0)
    def _init():
        out_ref[...] = jnp.zeros_like(out_ref)

    bvec = batch_ref[0, 0, :]
    gids = lax.broadcasted_iota(jnp.int32, (num_graphs, BLK), 0)
    onehot_t = jnp.where(gids == bvec[None, :], 1.0, 0.0)
    out_ref[...] += jnp.dot(onehot_t, feat_ref[...],
                            preferred_element_type=jnp.float32)


def _tc_bc_body(feat_ref, batch_ref, psc_ref, ptc_ref, vn_ref, w_ref, b_ref,
                out_ref, vnout_ref, vn_scr, *, num_graphs):
    i = pl.program_id(0)

    @pl.when(i == 0)
    def _fc():
        pool = psc_ref[0] + psc_ref[1] + ptc_ref[...]
        vn_tmp = pool + vn_ref[...]
        vn_o = jnp.maximum(
            jnp.dot(vn_tmp, w_ref[...], preferred_element_type=jnp.float32)
            + b_ref[...], 0.0) + vn_ref[...]
        vn_scr[...] = vn_o
        vnout_ref[...] = vn_o

    bvec = batch_ref[0, 0, :]
    gids = lax.broadcasted_iota(jnp.int32, (BLK, num_graphs), 1)
    onehot = jnp.where(gids == bvec[:, None], 1.0, 0.0)
    out_ref[...] = feat_ref[...] + jnp.dot(
        onehot, vn_scr[...], preferred_element_type=jnp.float32)


def kernel(feat, vn_feat, W, b, batch):
    n, d = feat.shape
    num_graphs = vn_feat.shape[0]
    n1 = N1_BLOCKS * BLK

    num_blocks = (n + BLK - 1) // BLK
    pad = num_blocks * BLK - n
    batch_r = jnp.pad(batch, (0, pad)).reshape(num_blocks, 1, BLK)

    p_sc = _sc_segsum(feat, batch, n1)

    p_tc = pl.pallas_call(
        functools.partial(_tc_pool_body, num_graphs=num_graphs),
        grid=(N1_BLOCKS,),
        in_specs=[
            pl.BlockSpec((BLK, d), lambda i: (i, 0)),
            pl.BlockSpec((1, 1, BLK), lambda i: (i, 0, 0)),
        ],
        out_specs=pl.BlockSpec((num_graphs, d), lambda i: (0, 0)),
        out_shape=jax.ShapeDtypeStruct((num_graphs, d), jnp.float32),
        compiler_params=pltpu.CompilerParams(
            dimension_semantics=("arbitrary",),
        ),
    )(feat, batch_r)

    feat_out, vn_out = pl.pallas_call(
        functools.partial(_tc_bc_body, num_graphs=num_graphs),
        grid=(num_blocks,),
        in_specs=[
            pl.BlockSpec((BLK, d), lambda i: (i, 0)),
            pl.BlockSpec((1, 1, BLK), lambda i: (i, 0, 0)),
            pl.BlockSpec((NC, num_graphs, d), lambda i: (0, 0, 0)),
            pl.BlockSpec((num_graphs, d), lambda i: (0, 0)),
            pl.BlockSpec((num_graphs, d), lambda i: (0, 0)),
            pl.BlockSpec((d, d), lambda i: (0, 0)),
            pl.BlockSpec((1, d), lambda i: (0, 0)),
        ],
        out_specs=[
            pl.BlockSpec((BLK, d), lambda i: (i, 0)),
            pl.BlockSpec((num_graphs, d), lambda i: (0, 0)),
        ],
        scratch_shapes=[
            pltpu.VMEM((num_graphs, d), jnp.float32),
        ],
        out_shape=(
            jax.ShapeDtypeStruct((n, d), jnp.float32),
            jax.ShapeDtypeStruct((num_graphs, d), jnp.float32),
        ),
        compiler_params=pltpu.CompilerParams(
            dimension_semantics=("arbitrary",),
        ),
    )(feat, batch_r, p_sc, p_tc, vn_feat, W, b.reshape(1, d))
    return (feat_out, vn_out)


# speedup vs baseline: 1.9368x; 1.0105x over previous
import functools

import jax
import jax.numpy as jnp
from jax import lax
from jax.experimental import pallas as pl
from jax.experimental.pallas import tpu as pltpu
from jax.experimental.pallas import tpu_sc as plsc

NC, NS, L = 2, 16, 16
NW = NC * NS
CHUNK = 128
NBUF = 3
POOL_ROWS = 272
DUMP = 256

BLK = 16384
N1_BLOCKS = 4


def _sc_segsum_body(feat_hbm, batch_hbm, out_hbm, rows_v, idx_v, zb_v, tmp_v,
                    pool_sh, sem0, sem1, sem2, *, row0, n_rows, nchunks,
                    per_w):
    c = lax.axis_index("c")
    s = lax.axis_index("s")
    w = s * NC + c
    sems = (sem0, sem1, sem2)

    for i in range(17):
        for j in range(8):
            zb_v[i, pl.ds(j * L, L)] = jnp.zeros((L,), jnp.float32)
    pltpu.sync_copy(zb_v, pool_sh.at[pl.ds(s * 17, 17)])
    plsc.subcore_barrier()

    def base_of(g):
        return row0 + jnp.minimum(g * CHUNK, (n_rows - row0) - CHUNK)

    def start(t, b):
        g = t * NW + w

        @pl.when(g < nchunks)
        def _():
            be = base_of(g)
            pltpu.async_copy(feat_hbm.at[pl.ds(be, CHUNK)], rows_v.at[b],
                             sems[b])
            pltpu.async_copy(batch_hbm.at[pl.ds(be, CHUNK)], idx_v.at[b],
                             sems[b])

    def finish(t, b):
        g = t * NW + w

        @pl.when(g < nchunks)
        def _():
            base = row0 + g * CHUNK
            be = base_of(g)
            pltpu.make_async_copy(feat_hbm.at[pl.ds(be, CHUNK)],
                                  rows_v.at[b], sems[b]).wait()
            pltpu.make_async_copy(batch_hbm.at[pl.ds(be, CHUNK)],
                                  idx_v.at[b], sems[b]).wait()
            lane = lax.broadcasted_iota(jnp.int32, (L,), 0)
            for j in range(CHUNK // L):
                rid = be + j * L + lane
                v = idx_v[b, pl.ds(j * L, L)]
                idx_v[b, pl.ds(j * L, L)] = jnp.where(rid >= base, v, DUMP)
            pltpu.sync_copy(rows_v.at[b], pool_sh.at[idx_v.at[b]], add=True)

    for b in range(NBUF):
        start(b, b)

    ngroups = (per_w + NBUF - 1) // NBUF

    def gbody(tg, carry):
        for b in range(NBUF):
            t = tg * NBUF + b
            finish(t, b)
            start(t + NBUF, b)
        return carry

    lax.fori_loop(0, ngroups, gbody, 0)
    plsc.subcore_barrier()

    pltpu.sync_copy(pool_sh.at[pl.ds(s * L, L)], tmp_v)
    pltpu.sync_copy(tmp_v, out_hbm.at[c, pl.ds(s * L, L)])


def _sc_segsum(feat, batch, row0):
    n, d = feat.shape
    nchunks = (n - row0 + CHUNK - 1) // CHUNK
    per_w = (nchunks + NW - 1) // NW
    mesh = plsc.VectorSubcoreMesh(core_axis_name="c", subcore_axis_name="s",
                                  num_cores=NC, num_subcores=NS)
    f = pl.kernel(
        functools.partial(_sc_segsum_body, row0=row0, n_rows=n,
                          nchunks=nchunks, per_w=per_w),
        out_type=jax.ShapeDtypeStruct((NC, NS * L, d), jnp.float32),
        mesh=mesh,
        scratch_types=[
            pltpu.VMEM((NBUF, CHUNK, d), jnp.float32),
            pltpu.VMEM((NBUF, CHUNK), jnp.int32),
            pltpu.VMEM((17, d), jnp.float32),
            pltpu.VMEM((L, d), jnp.float32),
            pltpu.VMEM_SHARED((POOL_ROWS, d), jnp.float32),
            pltpu.SemaphoreType.DMA,
            pltpu.SemaphoreType.DMA,
            pltpu.SemaphoreType.DMA,
        ],
    )
    return f(feat, batch)


def _tc_pool_body(feat_ref, batch_ref, out_ref, *, num_graphs):
    i = pl.program_id(0)

    @pl.when(i == 0)
    def _init():
        out_ref[...] = jnp.zeros_like(out_ref)

    bvec = batch_ref[0, 0, :]
    gids = lax.broadcasted_iota(jnp.int32, (num_graphs, BLK), 0)
    onehot_t = jnp.where(gids == bvec[None, :], 1.0, 0.0)
    out_ref[...] += jnp.dot(onehot_t, feat_ref[...],
                            preferred_element_type=jnp.float32)


def _tc_bc_body(feat_ref, batch_ref, psc_ref, ptc_ref, vn_ref, w_ref, b_ref,
                out_ref, vnout_ref, vn_scr, *, num_graphs):
    i = pl.program_id(0)

    @pl.when(i == 0)
    def _fc():
        pool = psc_ref[0] + psc_ref[1] + ptc_ref[...]
        vn_tmp = pool + vn_ref[...]
        vn_o = jnp.maximum(
            jnp.dot(vn_tmp, w_ref[...], preferred_element_type=jnp.float32)
            + b_ref[...], 0.0) + vn_ref[...]
        vn_scr[...] = vn_o
        vnout_ref[...] = vn_o

    bvec = batch_ref[0, 0, :]
    gids = lax.broadcasted_iota(jnp.int32, (BLK, num_graphs), 1)
    onehot = jnp.where(gids == bvec[:, None], 1.0, 0.0)
    out_ref[...] = feat_ref[...] + jnp.dot(
        onehot, vn_scr[...], preferred_element_type=jnp.float32)


def kernel(feat, vn_feat, W, b, batch):
    n, d = feat.shape
    num_graphs = vn_feat.shape[0]
    n1 = N1_BLOCKS * BLK

    num_blocks = (n + BLK - 1) // BLK
    pad = num_blocks * BLK - n
    batch_r = jnp.pad(batch, (0, pad)).reshape(num_blocks, 1, BLK)

    p_sc = _sc_segsum(feat, batch, n1)

    p_tc = pl.pallas_call(
        functools.partial(_tc_pool_body, num_graphs=num_graphs),
        grid=(N1_BLOCKS,),
        in_specs=[
            pl.BlockSpec((BLK, d), lambda i: (i, 0)),
            pl.BlockSpec((1, 1, BLK), lambda i: (i, 0, 0)),
        ],
        out_specs=pl.BlockSpec((num_graphs, d), lambda i: (0, 0)),
        out_shape=jax.ShapeDtypeStruct((num_graphs, d), jnp.float32),
        compiler_params=pltpu.CompilerParams(
            dimension_semantics=("arbitrary",),
        ),
    )(feat, batch_r)

    feat_out, vn_out = pl.pallas_call(
        functools.partial(_tc_bc_body, num_graphs=num_graphs),
        grid=(num_blocks,),
        in_specs=[
            pl.BlockSpec((BLK, d), lambda i: (i, 0)),
            pl.BlockSpec((1, 1, BLK), lambda i: (i, 0, 0)),
            pl.BlockSpec((NC, num_graphs, d), lambda i: (0, 0, 0)),
            pl.BlockSpec((num_graphs, d), lambda i: (0, 0)),
            pl.BlockSpec((num_graphs, d), lambda i: (0, 0)),
            pl.BlockSpec((d, d), lambda i: (0, 0)),
            pl.BlockSpec((1, d), lambda i: (0, 0)),
        ],
        out_specs=[
            pl.BlockSpec((BLK, d), lambda i: (i, 0)),
            pl.BlockSpec((num_graphs, d), lambda i: (0, 0)),
        ],
        scratch_shapes=[
            pltpu.VMEM((num_graphs, d), jnp.float32),
        ],
        out_shape=(
            jax.ShapeDtypeStruct((n, d), jnp.float32),
            jax.ShapeDtypeStruct((num_graphs, d), jnp.float32),
        ),
        compiler_params=pltpu.CompilerParams(
            dimension_semantics=("arbitrary",),
        ),
    )(feat, batch_r, p_sc, p_tc, vn_feat, W, b.reshape(1, d))
    return (feat_out, vn_out)
